# Initial kernel scaffold; baseline (speedup 1.0000x reference)
import jax
import jax.numpy as jnp
from jax.experimental import pallas as pl


def kernel(x, adj, W1, b1, W2, b2, lin_w, lin_b):
    raise NotImplementedError("write your pallas kernel here")


# speedup vs baseline: 1.0902x; 1.0902x over previous
import jax
import jax.numpy as jnp
from jax.experimental import pallas as pl
from jax.experimental.pallas import tpu as pltpu


def _pick_bm(n):
    for bm in (400, 200, 40, 16, 8):
        if n % bm == 0:
            return bm
    return n


def _layer1_body(adj_ref, x_ref, w1_ref, b1_ref, w2_ref, s2_ref):
    t = jnp.dot(adj_ref[...], x_ref[...], preferred_element_type=jnp.float32)
    z = jnp.dot(t, w1_ref[...], preferred_element_type=jnp.float32) + b1_ref[...]
    h = jnp.maximum(z, 0.1 * z)
    s2_ref[...] = jnp.dot(h, w2_ref[...], preferred_element_type=jnp.float32)


def _layer2_body(adj_ref, s2_ref, b2_ref, lw_ref, lb_ref, out_ref):
    z = jnp.dot(adj_ref[...], s2_ref[...], preferred_element_type=jnp.float32) + b2_ref[...]
    h = jnp.maximum(z, 0.1 * z)
    out_ref[...] = jnp.dot(h, lw_ref[...], preferred_element_type=jnp.float32) + lb_ref[...]


def kernel(x, adj, W1, b1, W2, b2, lin_w, lin_b):
    n, nfeat = x.shape
    h1 = W1.shape[1]
    h2 = W2.shape[1]
    ncls = lin_w.shape[1]
    ncls_pad = ((ncls + 127) // 128) * 128

    bm = _pick_bm(n)
    grid = (n // bm,)

    b1r = b1.reshape(1, h1)
    b2r = b2.reshape(1, h2)
    lw = jnp.pad(lin_w, ((0, 0), (0, ncls_pad - ncls)))
    lb = jnp.pad(lin_b, (0, ncls_pad - ncls)).reshape(1, ncls_pad)

    const = lambda shape: pl.BlockSpec(shape, lambda m: (0, 0))
    rowblk = lambda w: pl.BlockSpec((bm, w), lambda m: (m, 0))

    s2 = pl.pallas_call(
        _layer1_body,
        grid=grid,
        in_specs=[
            rowblk(n),
            const((n, nfeat)),
            const((nfeat, h1)),
            const((1, h1)),
            const((h1, h2)),
        ],
        out_specs=rowblk(h2),
        out_shape=jax.ShapeDtypeStruct((n, h2), jnp.float32),
        compiler_params=pltpu.CompilerParams(
            dimension_semantics=("parallel",),
        ),
    )(adj, x, W1, b1r, W2)

    out = pl.pallas_call(
        _layer2_body,
        grid=grid,
        in_specs=[
            rowblk(n),
            const((n, h2)),
            const((1, h2)),
            const((h2, ncls_pad)),
            const((1, ncls_pad)),
        ],
        out_specs=rowblk(ncls_pad),
        out_shape=jax.ShapeDtypeStruct((n, ncls_pad), jnp.float32),
        compiler_params=pltpu.CompilerParams(
            dimension_semantics=("parallel",),
        ),
    )(adj, s2, b2r, lw, lb)

    return out[:, :ncls]
